# baseline (speedup 1.0000x reference)
import jax
import jax.numpy as jnp
from jax.experimental import pallas as pl
from jax.experimental.pallas import tpu as pltpu

_G = 76
_GG = _G * _G
_NUM = 3
_CP = 14
_NCLS = 7
_STRIDE = 8.0


def _sigmoid(v):
    return 1.0 / (1.0 + jnp.exp(-v))


def _arctan(z):
    az = jnp.abs(z)
    inv = az > 1.0
    u = jnp.where(inv, 1.0 / az, az)
    u2 = u * u
    p = u * (0.9998660 + u2 * (-0.3302995 + u2 * (
        0.1801410 + u2 * (-0.0851330 + u2 * 0.0208351))))
    r = jnp.where(inv, (jnp.pi / 2.0) - p, p)
    return jnp.sign(z) * r


def _decode_body(anchors_ref, x_ref, out_ref):
    xb = x_ref[0]
    p = jax.lax.broadcasted_iota(jnp.int32, (1, _GG), 1)
    ii = p // _G
    jj = p - ii * _G
    gx = jj.astype(jnp.float32)
    gy = ii.astype(jnp.float32)
    eye = jnp.eye(16, dtype=jnp.float32)
    for a in range(_NUM):
        b0 = a * _CP
        im = xb[b0 + 4:b0 + 5, :]
        re_ = xb[b0 + 5:b0 + 6, :]
        yaw = _arctan(im / re_)
        conf = _sigmoid(xb[b0 + 6:b0 + 7, :])
        ax = jnp.floor((_sigmoid(xb[b0 + 0:b0 + 1, :]) + gx) * _STRIDE)
        ay = jnp.floor((_sigmoid(xb[b0 + 1:b0 + 2, :]) + gy) * _STRIDE)
        aw = jnp.exp(xb[b0 + 2:b0 + 3, :]) * anchors_ref[a, 0]
        ah = jnp.exp(xb[b0 + 3:b0 + 4, :]) * anchors_ref[a, 1]
        rows = [im, re_, yaw, conf, ax, ay, aw, ah]
        for k in range(_NCLS):
            m = 3 * k + a
            c = (m // _NCLS) * _CP + _NCLS + (m % _NCLS)
            rows.append(xb[c:c + 1, :])
        rows.append(jnp.zeros((1, _GG), jnp.float32))
        f = jnp.concatenate(rows, axis=0)
        t = jax.lax.dot_general(
            f, eye, (((0,), (0,)), ((), ())),
            preferred_element_type=jnp.float32)
        out_ref[0, a * _GG:(a + 1) * _GG, :] = t[:, :15]


def kernel(x, anchors):
    B = x.shape[0]
    xr = x.reshape(B, _NUM * _CP, _GG)
    return pl.pallas_call(
        _decode_body,
        grid=(B,),
        in_specs=[
            pl.BlockSpec(memory_space=pltpu.SMEM),
            pl.BlockSpec((1, _NUM * _CP, _GG), lambda b: (b, 0, 0)),
        ],
        out_specs=pl.BlockSpec((1, _NUM * _GG, 15), lambda b: (b, 0, 0)),
        out_shape=jax.ShapeDtypeStruct((B, _NUM * _GG, 15), jnp.float32),
    )(anchors, xr)
